# baseline (speedup 1.0000x reference)
import functools

import jax
import jax.numpy as jnp
from jax import lax
from jax.experimental import pallas as pl
from jax.experimental.pallas import tpu as pltpu
from jax.experimental.pallas import tpu_sc as plsc

BATCH = 16384
D = 64
NC = 2
NS = 16
NW = NC * NS
BPW = BATCH // NW
C = 128
NCH = BPW // C

_mesh = plsc.VectorSubcoreMesh(core_axis_name="c", subcore_axis_name="s")


@functools.partial(
    pl.kernel,
    mesh=_mesh,
    out_type=jax.ShapeDtypeStruct((BATCH // 8, 128), jnp.float32),
    scratch_types=[
        pltpu.VMEM((NCH, C), jnp.int32),
        pltpu.VMEM((NCH, C), jnp.int32),
        pltpu.VMEM((NCH, C), jnp.int32),
        pltpu.VMEM((C, 2 * D), jnp.float32),
        pltpu.VMEM((C, 2 * D), jnp.float32),
        pltpu.VMEM((C, 2 * D), jnp.float32),
        pltpu.VMEM((C, 2 * D), jnp.float32),
        pltpu.VMEM((C, 2 * D), jnp.float32),
        pltpu.VMEM((C, 2 * D), jnp.float32),
        pltpu.VMEM((C * 16,), jnp.float32),
        pltpu.VMEM((C * 16,), jnp.float32),
        pltpu.VMEM((C * 16,), jnp.float32),
        pltpu.VMEM((C // 8, 128), jnp.float32),
        pltpu.SemaphoreType.DMA,
    ],
)
def _complex_partial_kernel(h_hbm, r_hbm, t_hbm, mh_hbm, mr_hbm, mt_hbm,
                            ere_hbm, eim_hbm, rre_hbm, rim_hbm, out_hbm,
                            hi_v, ri_v, ti_v, hre_v, him_v, tre_v, tim_v,
                            rre_v, rim_v, mh_v, mr_v, mt_v, pacc_v, sem):
    cid = lax.axis_index("c")
    sid = lax.axis_index("s")
    wid = sid * NC + cid

    pltpu.sync_copy(h_hbm.at[wid], hi_v)
    pltpu.sync_copy(r_hbm.at[wid], ri_v)
    pltpu.sync_copy(t_hbm.at[wid], ti_v)

    for ch in range(NCH):
        mbase = (wid * BPW + ch * C) * 16
        cp1 = pltpu.async_copy(ere_hbm.at[hi_v.at[ch]], hre_v, sem)
        cp2 = pltpu.async_copy(eim_hbm.at[hi_v.at[ch]], him_v, sem)
        cp3 = pltpu.async_copy(ere_hbm.at[ti_v.at[ch]], tre_v, sem)
        cp4 = pltpu.async_copy(eim_hbm.at[ti_v.at[ch]], tim_v, sem)
        cp5 = pltpu.async_copy(rre_hbm.at[ri_v.at[ch]], rre_v, sem)
        cp6 = pltpu.async_copy(rim_hbm.at[ri_v.at[ch]], rim_v, sem)
        pltpu.sync_copy(mh_hbm.at[pl.ds(mbase, C * 16)], mh_v)
        pltpu.sync_copy(mr_hbm.at[pl.ds(mbase, C * 16)], mr_v)
        pltpu.sync_copy(mt_hbm.at[pl.ds(mbase, C * 16)], mt_v)
        cp1.wait()
        cp2.wait()
        cp3.wait()
        cp4.wait()
        cp5.wait()
        cp6.wait()

        def row_body(row, carry):
            moff = row * 16
            mh = mh_v[pl.ds(moff, 16)]
            mr = mr_v[pl.ds(moff, 16)]
            mt = mt_v[pl.ds(moff, 16)]
            acc = jnp.zeros((16,), jnp.float32)
            for j in range(D // 16):
                lo = pl.ds(j * 16, 16)
                hi = pl.ds(D + j * 16, 16)
                a0 = hre_v[row, lo]
                a = a0 + mh * (hre_v[row, hi] - a0)
                b0 = him_v[row, lo]
                b = b0 + mh * (him_v[row, hi] - b0)
                c0 = tre_v[row, lo]
                c = c0 + mt * (tre_v[row, hi] - c0)
                d0 = tim_v[row, lo]
                d = d0 + mt * (tim_v[row, hi] - d0)
                p0 = rre_v[row, lo]
                p = p0 + mr * (rre_v[row, hi] - p0)
                q0 = rim_v[row, lo]
                q = q0 + mr * (rim_v[row, hi] - q0)
                acc = acc + p * (a * c + b * d) + q * (a * d - b * c)
            pacc_v[row // 8, pl.ds((row % 8) * 16, 16)] = acc
            return carry

        lax.fori_loop(0, C, row_body, 0)

        pltpu.sync_copy(pacc_v,
                        out_hbm.at[pl.ds(wid * (BPW // 8) + ch * (C // 8),
                                         C // 8)])


def _reduce_body(x_ref, o_ref):
    x = x_ref[...]
    rows = lax.broadcasted_iota(jnp.int32, (128, 128), 0)
    cols = lax.broadcasted_iota(jnp.int32, (128, 128), 1)
    m = (rows // 16 == cols).astype(jnp.float32)
    o_ref[...] = -jnp.dot(x, m, preferred_element_type=jnp.float32)


_reduce_call = pl.pallas_call(
    _reduce_body,
    out_shape=jax.ShapeDtypeStruct((BATCH // 8, 128), jnp.float32),
)


def kernel(h, r, t, entity_re, entity_im, relation_re, relation_im):
    h = h.astype(jnp.int32)
    r = r.astype(jnp.int32)
    t = t.astype(jnp.int32)
    h3 = (h >> 1).reshape(NW, NCH, C)
    r3 = (r >> 1).reshape(NW, NCH, C)
    t3 = (t >> 1).reshape(NW, NCH, C)
    mh = jnp.broadcast_to((h & 1).astype(jnp.float32)[:, None],
                          (BATCH, 16)).reshape(-1)
    mr = jnp.broadcast_to((r & 1).astype(jnp.float32)[:, None],
                          (BATCH, 16)).reshape(-1)
    mt = jnp.broadcast_to((t & 1).astype(jnp.float32)[:, None],
                          (BATCH, 16)).reshape(-1)
    ere2 = entity_re.reshape(-1, 2 * D)
    eim2 = entity_im.reshape(-1, 2 * D)
    rre2 = relation_re.reshape(-1, 2 * D)
    rim2 = relation_im.reshape(-1, 2 * D)
    partial = _complex_partial_kernel(h3, r3, t3, mh, mr, mt,
                                      ere2, eim2, rre2, rim2)
    return _reduce_call(partial)[:, :8].reshape(BATCH)
